# baseline (speedup 1.0000x reference)
import numpy as np
import jax
import jax.numpy as jnp
from jax.experimental import pallas as pl
from jax.experimental.pallas import tpu as pltpu

_NEIGHBORS = 2
_TW = 10
_T_RES = 100
_NX = 64
_NY = 64
_B = 4
_TMIN, _TMAX = 0.0, 1.0
_LX, _LY = 1.0, 1.0
_N = _NX * _NY


def _linspace_f32(start, stop, num):
    i = np.arange(num - 1, dtype=np.float32) / np.float32(num - 1)
    start = np.float32(start)
    stop = np.float32(stop)
    body = start * (np.float32(1.0) - i) + stop * i
    return np.concatenate([body, np.array([stop], dtype=np.float32)])


def _static_graph():
    x_np = _linspace_f32(0.0, _LX, _NX)
    y_np = _linspace_f32(0.0, _LY, _NY)
    dx = x_np[1] - x_np[0]
    dy = y_np[1] - y_np[0]
    radius = np.float32(_NEIGHBORS) * np.sqrt(dx ** 2 + dy ** 2, dtype=np.float32) + np.float32(0.0001)
    gx_np, gy_np = np.meshgrid(x_np, y_np, indexing="ij")
    grid_np = np.stack((gx_np, gy_np), axis=2).astype(np.float32).reshape(-1, 2)
    d2 = np.sum((grid_np[:, None, :] - grid_np[None, :, :]) ** 2, axis=-1, dtype=np.float32)
    mask = (d2 <= radius ** 2) & (~np.eye(_N, dtype=bool))
    src_np, dst_np = np.nonzero(mask)
    edges = [np.stack([src_np + b * _N, dst_np + b * _N], axis=0) for b in range(_B)]
    edge_index = np.concatenate(edges, axis=1).astype(np.int32)
    t_table = _linspace_f32(_TMIN, _TMAX, _T_RES)
    return grid_np, edge_index, t_table


_GRID_NP, _EDGE_INDEX_NP, _T_TABLE_NP = _static_graph()


def _body(steps_ref, t_ref, data_ref, labels_ref, gridT_ref,
          uT_ref, yT_ref, posT_ref, batch_ref):
    for b in range(_B):
        uT_ref[:, b * _N:(b + 1) * _N] = data_ref[b].reshape(_TW, _N)
        yT_ref[:, b * _N:(b + 1) * _N] = labels_ref[b].reshape(_TW, _N)
        tval = t_ref[0, steps_ref[0, b]]
        posT_ref[0:1, b * _N:(b + 1) * _N] = jnp.full((1, _N), tval, jnp.float32)
        posT_ref[1:3, b * _N:(b + 1) * _N] = gridT_ref[...]
        batch_ref[pl.ds(b * 32, 32), :] = jnp.full((32, 128), b, jnp.int32)


def kernel(data, labels, steps):
    steps2 = steps.reshape(1, _B).astype(jnp.int32)
    t2 = jnp.asarray(_T_TABLE_NP).reshape(1, _T_RES)
    gridT = jnp.asarray(np.ascontiguousarray(_GRID_NP.T))

    uT, yT, posT, batch2d = pl.pallas_call(
        _body,
        in_specs=[
            pl.BlockSpec(memory_space=pltpu.SMEM),
            pl.BlockSpec(memory_space=pltpu.SMEM),
            pl.BlockSpec((_B, _TW, _NX, _NY), lambda: (0, 0, 0, 0)),
            pl.BlockSpec((_B, _TW, _NX, _NY), lambda: (0, 0, 0, 0)),
            pl.BlockSpec((2, _N), lambda: (0, 0)),
        ],
        out_specs=[
            pl.BlockSpec((_TW, _B * _N), lambda: (0, 0)),
            pl.BlockSpec((_TW, _B * _N), lambda: (0, 0)),
            pl.BlockSpec((3, _B * _N), lambda: (0, 0)),
            pl.BlockSpec((128, 128), lambda: (0, 0)),
        ],
        out_shape=[
            jax.ShapeDtypeStruct((_TW, _B * _N), jnp.float32),
            jax.ShapeDtypeStruct((_TW, _B * _N), jnp.float32),
            jax.ShapeDtypeStruct((3, _B * _N), jnp.float32),
            jax.ShapeDtypeStruct((128, 128), jnp.int32),
        ],
    )(steps2, t2, data, labels, gridT)

    u_new = uT.T
    y_new = yT.T
    pos = posT.T
    batch = batch2d.reshape(_B * _N)
    edge16 = jax.lax.optimization_barrier(jnp.asarray(_EDGE_INDEX_NP.astype(np.int16)))
    edge_index = edge16.astype(jnp.int32)
    return u_new, edge_index, y_new, pos, batch


# speedup vs baseline: 1.4882x; 1.4882x over previous
import numpy as np
import jax
import jax.numpy as jnp
from jax.experimental import pallas as pl
from jax.experimental.pallas import tpu as pltpu

_NEIGHBORS = 2
_TW = 10
_T_RES = 100
_NX = 64
_NY = 64
_B = 4
_TMIN, _TMAX = 0.0, 1.0
_LX, _LY = 1.0, 1.0
_N = _NX * _NY
_E = 378000
_EBLK = 192000


def _linspace_f32(start, stop, num):
    i = np.arange(num - 1, dtype=np.float32) / np.float32(num - 1)
    start = np.float32(start)
    stop = np.float32(stop)
    body = start * (np.float32(1.0) - i) + stop * i
    return np.concatenate([body, np.array([stop], dtype=np.float32)])


def _static_graph():
    x_np = _linspace_f32(0.0, _LX, _NX)
    y_np = _linspace_f32(0.0, _LY, _NY)
    dx = x_np[1] - x_np[0]
    dy = y_np[1] - y_np[0]
    radius = np.float32(_NEIGHBORS) * np.sqrt(dx ** 2 + dy ** 2, dtype=np.float32) + np.float32(0.0001)
    gx_np, gy_np = np.meshgrid(x_np, y_np, indexing="ij")
    grid_np = np.stack((gx_np, gy_np), axis=2).astype(np.float32).reshape(-1, 2)
    d2 = np.sum((grid_np[:, None, :] - grid_np[None, :, :]) ** 2, axis=-1, dtype=np.float32)
    mask = (d2 <= radius ** 2) & (~np.eye(_N, dtype=bool))
    src_np, dst_np = np.nonzero(mask)
    edges = [np.stack([src_np + b * _N, dst_np + b * _N], axis=0) for b in range(_B)]
    edge_index = np.concatenate(edges, axis=1).astype(np.int32)
    t_table = _linspace_f32(_TMIN, _TMAX, _T_RES)
    return grid_np, edge_index, t_table


_GRID_NP, _EDGE_INDEX_NP, _T_TABLE_NP = _static_graph()


def _body(steps_ref, t_ref, data_ref, labels_ref, gridT_ref, edge16_ref,
          uT_ref, yT_ref, posT_ref, batch_ref, edge_ref):
    r = pl.program_id(0)
    edge_ref[...] = edge16_ref[...].astype(jnp.int32)
    for bb in range(2):
        b = 2 * r + bb
        uT_ref[:, bb * _N:(bb + 1) * _N] = data_ref[bb].reshape(_TW, _N)
        yT_ref[:, bb * _N:(bb + 1) * _N] = labels_ref[bb].reshape(_TW, _N)
        tval = t_ref[0, steps_ref[0, b]]
        posT_ref[0:1, bb * _N:(bb + 1) * _N] = jnp.full((1, _N), tval, jnp.float32)
        posT_ref[1:3, bb * _N:(bb + 1) * _N] = gridT_ref[...]
        batch_ref[pl.ds(bb * 32, 32), :] = jnp.full((32, 128), b, jnp.int32)


def kernel(data, labels, steps):
    steps2 = steps.reshape(1, _B).astype(jnp.int32)
    t2 = jnp.asarray(_T_TABLE_NP).reshape(1, _T_RES)
    gridT = jnp.asarray(np.ascontiguousarray(_GRID_NP.T))
    edge16 = jnp.asarray(_EDGE_INDEX_NP.astype(np.int16))

    uT, yT, posT, batch2d, edge_index = pl.pallas_call(
        _body,
        grid=(2,),
        in_specs=[
            pl.BlockSpec(memory_space=pltpu.SMEM),
            pl.BlockSpec(memory_space=pltpu.SMEM),
            pl.BlockSpec((2, _TW, _NX, _NY), lambda r: (r, 0, 0, 0)),
            pl.BlockSpec((2, _TW, _NX, _NY), lambda r: (r, 0, 0, 0)),
            pl.BlockSpec((2, _N), lambda r: (0, 0)),
            pl.BlockSpec((2, _EBLK), lambda r: (0, r)),
        ],
        out_specs=[
            pl.BlockSpec((_TW, 2 * _N), lambda r: (0, r)),
            pl.BlockSpec((_TW, 2 * _N), lambda r: (0, r)),
            pl.BlockSpec((3, 2 * _N), lambda r: (0, r)),
            pl.BlockSpec((64, 128), lambda r: (r, 0)),
            pl.BlockSpec((2, _EBLK), lambda r: (0, r)),
        ],
        out_shape=[
            jax.ShapeDtypeStruct((_TW, _B * _N), jnp.float32),
            jax.ShapeDtypeStruct((_TW, _B * _N), jnp.float32),
            jax.ShapeDtypeStruct((3, _B * _N), jnp.float32),
            jax.ShapeDtypeStruct((128, 128), jnp.int32),
            jax.ShapeDtypeStruct((2, _E), jnp.int32),
        ],
        compiler_params=pltpu.CompilerParams(
            dimension_semantics=("arbitrary",),
        ),
    )(steps2, t2, data, labels, gridT, edge16)

    u_new = uT.T
    y_new = yT.T
    pos = posT.T
    batch = batch2d.reshape(_B * _N)
    return u_new, edge_index, y_new, pos, batch
